# baseline (speedup 1.0000x reference)
import functools

import jax
import jax.numpy as jnp
from jax import lax
from jax.experimental import pallas as pl
from jax.experimental.pallas import tpu as pltpu
from jax.experimental.pallas import tpu_sc as plsc

NUM_SLOTS = 20000


def _sc_body(M, L, ROWS, data_hbm, idx_hbm, table_s, table_t, data_v, idx_v):
    SENT = L
    ZFUSED = (L + 1) * 32 + (L + 1)
    NC = 2
    wid = lax.axis_index("s") * NC + lax.axis_index("c")
    base = wid * ROWS

    sent_vec = jnp.full((16,), SENT, jnp.int32)
    CH = M // 16

    def init(i, carry):
        table_s[pl.ds(i * 16, 16)] = sent_vec
        table_t[pl.ds(i * 16, 16)] = sent_vec
        return carry

    lax.fori_loop(0, NUM_SLOTS // 16, init, 0)

    def row_step(r, carry):
        row = base + r
        pltpu.sync_copy(data_hbm.at[row], data_v)

        def chunk(c):
            w = data_v[pl.ds(c * 16, 16)]
            return w >> 5, w & 31

        def passes(dirty_in):
            for c in range(2 * CH):
                tbl = table_s if c < CH else table_t
                k, p = chunk(c)
                g = plsc.load_gather(tbl, [k])
                plsc.store_scatter(tbl, [k], p, mask=p < g)

            dirty = jnp.zeros((16,), jnp.bool_)
            for c in range(2 * CH):
                own_t, cross_t = ((table_s, table_t) if c < CH
                                  else (table_t, table_s))
                k, p = chunk(c)
                own = plsc.load_gather(own_t, [k])
                cross = plsc.load_gather(cross_t, [k])
                dirty = dirty | (p < own)
                fused = jnp.where(p < SENT, own * 32 + cross, ZFUSED)
                idx_v[pl.ds(c * 16, 16)] = fused
            return jnp.any(dirty)

        lax.while_loop(lambda d: d, passes, jnp.bool_(True))

        for c in range(2 * CH):
            tbl = table_s if c < CH else table_t
            plsc.store_scatter(tbl, [chunk(c)[0]], sent_vec)

        pltpu.sync_copy(idx_v, idx_hbm.at[row])
        return carry

    lax.fori_loop(0, ROWS, row_step, 0)


def _tc_body(M, R, fused_col_ref, emb_ref, src_out_ref, tgt_out_ref):
    emb = emb_ref[...]
    col_blk = fused_col_ref[...]
    jcol = lax.broadcasted_iota(jnp.int32, (2 * M, 64), 1)
    for r in range(R):
        fused = col_blk[r]
        own = fused >> 5
        cross = fused & 31
        oh = ((jcol == own) | (jcol == cross + 32)).astype(jnp.float32)
        out = jnp.dot(oh, emb, preferred_element_type=jnp.float32)
        src_out_ref[r] = out[0:M, :]
        tgt_out_ref[r] = out[M:2 * M, :]


def kernel(src_walks, tgt_walks, src_lens, tgt_lens, own_emb, cross_emb):
    B, K, L = src_walks.shape
    M = K * L
    HALF = own_emb.shape[1]
    POS_DIM = HALF + cross_emb.shape[1]
    SENT = L
    NW = 32
    ROWS = B // NW

    src_walks = src_walks.astype(jnp.int32)
    tgt_walks = tgt_walks.astype(jnp.int32)
    pos_grid = jnp.arange(L, dtype=jnp.int32).reshape(1, 1, L)
    src_valid = (pos_grid < src_lens[..., None]) & (src_walks != 0)
    tgt_valid = (pos_grid < tgt_lens[..., None]) & (tgt_walks != 0)
    pos_flat = jnp.broadcast_to(
        jnp.tile(jnp.arange(L, dtype=jnp.int32), K).reshape(1, M), (B, M))

    cat_data = jnp.concatenate(
        [src_walks.reshape(B, M) * 32
         + jnp.where(src_valid.reshape(B, M), pos_flat, SENT),
         tgt_walks.reshape(B, M) * 32
         + jnp.where(tgt_valid.reshape(B, M), pos_flat, SENT)], axis=1)

    mesh = plsc.VectorSubcoreMesh(core_axis_name="c", subcore_axis_name="s")
    fused = pl.kernel(
        functools.partial(_sc_body, M, L, ROWS),
        mesh=mesh,
        compiler_params=pltpu.CompilerParams(
            needs_layout_passes=False, use_tc_tiling_on_sc=False),
        out_type=jax.ShapeDtypeStruct((B, 2 * M), jnp.int32),
        scratch_types=[
            pltpu.VMEM((NUM_SLOTS,), jnp.int32),
            pltpu.VMEM((NUM_SLOTS,), jnp.int32),
            pltpu.VMEM((2 * M,), jnp.int32),
            pltpu.VMEM((2 * M,), jnp.int32),
        ],
    )(cat_data)

    emb_mat = (jnp.zeros((64, POS_DIM), jnp.float32)
               .at[0:L + 1, 0:HALF].set(own_emb[:L + 1])
               .at[32:32 + L + 1, HALF:POS_DIM].set(cross_emb[:L + 1]))

    R = 8
    src_pos, tgt_pos = pl.pallas_call(
        functools.partial(_tc_body, M, R),
        grid=(B // R,),
        in_specs=[
            pl.BlockSpec((R, 2 * M, 1), lambda i: (i, 0, 0)),
            pl.BlockSpec((64, POS_DIM), lambda i: (0, 0)),
        ],
        out_specs=[
            pl.BlockSpec((R, M, POS_DIM), lambda i: (i, 0, 0)),
            pl.BlockSpec((R, M, POS_DIM), lambda i: (i, 0, 0)),
        ],
        out_shape=[
            jax.ShapeDtypeStruct((B, M, POS_DIM), jnp.float32),
            jax.ShapeDtypeStruct((B, M, POS_DIM), jnp.float32),
        ],
    )(fused[..., None], emb_mat)

    return (src_pos.reshape(B, K, L, POS_DIM),
            tgt_pos.reshape(B, K, L, POS_DIM))


# speedup vs baseline: 2.0859x; 2.0859x over previous
import functools

import jax
import jax.numpy as jnp
from jax import lax
from jax.experimental import pallas as pl
from jax.experimental.pallas import tpu as pltpu
from jax.experimental.pallas import tpu_sc as plsc

NUM_SLOTS = 20000


def _sc_body(M, L, ROWS, data_hbm, idx_hbm, table_s, table_t, data_v, idx_v):
    SENT = L
    ZFUSED = (L + 1) * 32 + (L + 1)
    NC = 2
    wid = lax.axis_index("s") * NC + lax.axis_index("c")
    base = wid * ROWS

    sent_vec = jnp.full((16,), SENT, jnp.int32)
    CH = M // 16

    def init(i, carry):
        table_s[pl.ds(i * 16, 16)] = sent_vec
        table_t[pl.ds(i * 16, 16)] = sent_vec
        return carry

    lax.fori_loop(0, NUM_SLOTS // 16, init, 0)

    def row_step(r, carry):
        row = base + r
        pltpu.sync_copy(data_hbm.at[row], data_v)

        def chunk(c):
            w = data_v[pl.ds(c * 16, 16)]
            return w >> 5, w & 31

        def passes(dirty_in):
            for c in range(2 * CH):
                tbl = table_s if c < CH else table_t
                k, p = chunk(c)
                g = plsc.load_gather(tbl, [k])
                plsc.store_scatter(tbl, [k], p, mask=p < g)

            dirty = jnp.zeros((16,), jnp.bool_)
            for c in range(2 * CH):
                own_t, cross_t = ((table_s, table_t) if c < CH
                                  else (table_t, table_s))
                k, p = chunk(c)
                own = plsc.load_gather(own_t, [k])
                cross = plsc.load_gather(cross_t, [k])
                dirty = dirty | (p < own)
                fused = jnp.where(p < SENT, own * 32 + cross, ZFUSED)
                idx_v[pl.ds(c * 16, 16)] = fused
            return jnp.any(dirty)

        lax.while_loop(lambda d: d, passes, jnp.bool_(True))

        for c in range(2 * CH):
            tbl = table_s if c < CH else table_t
            plsc.store_scatter(tbl, [chunk(c)[0]], sent_vec)

        pltpu.sync_copy(idx_v, idx_hbm.at[row])
        return carry

    lax.fori_loop(0, ROWS, row_step, 0)


def _tc_body(M, R, fused_ref, emb_ref, src_out_ref, tgt_out_ref):
    emb = emb_ref[...]
    fused = fused_ref[...][0]
    own = fused >> 5
    cross = (fused & 31) + 32
    jrow = lax.broadcasted_iota(jnp.int32, (64, R * 2 * M), 0)
    ohT = ((jrow == own) | (jrow == cross)).astype(jnp.float32)
    out = lax.dot_general(ohT, emb, (((0,), (0,)), ((), ())),
                          preferred_element_type=jnp.float32)
    for r in range(R):
        src_out_ref[r] = out[r * 2 * M:r * 2 * M + M, :]
        tgt_out_ref[r] = out[r * 2 * M + M:(r + 1) * 2 * M, :]


def kernel(src_walks, tgt_walks, src_lens, tgt_lens, own_emb, cross_emb):
    B, K, L = src_walks.shape
    M = K * L
    HALF = own_emb.shape[1]
    POS_DIM = HALF + cross_emb.shape[1]
    SENT = L
    NW = 32
    ROWS = B // NW

    src_walks = src_walks.astype(jnp.int32)
    tgt_walks = tgt_walks.astype(jnp.int32)
    pos_grid = jnp.arange(L, dtype=jnp.int32).reshape(1, 1, L)
    src_valid = (pos_grid < src_lens[..., None]) & (src_walks != 0)
    tgt_valid = (pos_grid < tgt_lens[..., None]) & (tgt_walks != 0)
    pos_flat = jnp.broadcast_to(
        jnp.tile(jnp.arange(L, dtype=jnp.int32), K).reshape(1, M), (B, M))

    cat_data = jnp.concatenate(
        [src_walks.reshape(B, M) * 32
         + jnp.where(src_valid.reshape(B, M), pos_flat, SENT),
         tgt_walks.reshape(B, M) * 32
         + jnp.where(tgt_valid.reshape(B, M), pos_flat, SENT)], axis=1)

    mesh = plsc.VectorSubcoreMesh(core_axis_name="c", subcore_axis_name="s")
    fused = pl.kernel(
        functools.partial(_sc_body, M, L, ROWS),
        mesh=mesh,
        compiler_params=pltpu.CompilerParams(
            needs_layout_passes=False, use_tc_tiling_on_sc=False),
        out_type=jax.ShapeDtypeStruct((B, 2 * M), jnp.int32),
        scratch_types=[
            pltpu.VMEM((NUM_SLOTS,), jnp.int32),
            pltpu.VMEM((NUM_SLOTS,), jnp.int32),
            pltpu.VMEM((2 * M,), jnp.int32),
            pltpu.VMEM((2 * M,), jnp.int32),
        ],
    )(cat_data)

    emb_mat = (jnp.zeros((64, POS_DIM), jnp.float32)
               .at[0:L + 1, 0:HALF].set(own_emb[:L + 1])
               .at[32:32 + L + 1, HALF:POS_DIM].set(cross_emb[:L + 1]))

    R = 8
    src_pos, tgt_pos = pl.pallas_call(
        functools.partial(_tc_body, M, R),
        grid=(B // R,),
        in_specs=[
            pl.BlockSpec((1, 1, R * 2 * M), lambda i: (i, 0, 0)),
            pl.BlockSpec((64, POS_DIM), lambda i: (0, 0)),
        ],
        out_specs=[
            pl.BlockSpec((R, M, POS_DIM), lambda i: (i, 0, 0)),
            pl.BlockSpec((R, M, POS_DIM), lambda i: (i, 0, 0)),
        ],
        out_shape=[
            jax.ShapeDtypeStruct((B, M, POS_DIM), jnp.float32),
            jax.ShapeDtypeStruct((B, M, POS_DIM), jnp.float32),
        ],
    )(fused.reshape(B // R, 1, R * 2 * M), emb_mat)

    return (src_pos.reshape(B, K, L, POS_DIM),
            tgt_pos.reshape(B, K, L, POS_DIM))
